# baseline (speedup 1.0000x reference)
import functools

import jax
import jax.numpy as jnp
from jax import lax
from jax.experimental import pallas as pl
from jax.experimental.pallas import tpu as pltpu
from jax.experimental.pallas import tpu_sc as plsc

_B, _T, _N = 1024, 20, 2048
_D, _H, _FDIM, _QH = 64, 64, 1000, 256
_S = 32
_SC = 8
_R = _S * _T
_GRID = _B // _S


def _main_body(x_ref, fg_ref,
               wq0, bq0, wq1, bq1,
               we1a, we1b, be1, we2, be2, ge, bge, we3, be3,
               wn1a, wn1c, bn1, wn2, bn2, gn, bgn, wn3, bn3,
               wp0, bp0, wp1, bp1,
               out_ref, kl_ref):
    f32 = jnp.float32

    xb = x_ref[...].reshape(_R, _N)
    nrm = jnp.sqrt(jnp.sum(xb * xb, axis=1, keepdims=True))
    inv = 1.0 / jnp.maximum(nrm, 1e-6)
    h = jnp.dot(xb, wq0[...], preferred_element_type=f32)
    h = jnp.tanh(h * inv + bq0[...])
    h = jnp.dot(h, wq1[...], preferred_element_type=f32) + bq1[...]
    mu = h[:, :_D]
    logvar = h[:, _D:]
    klp = jnp.sum(-logvar + jnp.exp(logvar) + mu * mu - 1.0)
    kl_ref[...] = jnp.broadcast_to(klp.reshape(1, 1, 1), (1, 1, 128))

    node = mu
    a = jnp.dot(node, we1a[...], preferred_element_type=f32)
    b = jnp.dot(node, we1b[...], preferred_element_type=f32)
    aa = a.astype(jnp.bfloat16)
    bb = (b + be1[...]).astype(jnp.bfloat16)
    e = jax.nn.relu(aa.reshape(_S, _T, 1, _H) + bb.reshape(_S, 1, _T, _H))
    e = e.reshape(_S * _T * _T, _H)
    jmat = jnp.full((_H, _H), 1.0 / _H, dtype=f32)

    def _edge_tail(t):
        t = jnp.dot(t, we2[...], preferred_element_type=f32) + be2[...]
        t = t - jnp.dot(t, jmat, preferred_element_type=f32)
        vv = jnp.dot(t * t, jmat, preferred_element_type=f32)
        t = t * (ge[...] * lax.rsqrt(vv + 1e-5)) + bge[...]
        return jax.nn.relu(t)

    e = _edge_tail(e).reshape(_S, _T, _T, _H)

    diag = _edge_tail(jax.nn.relu(aa + bb))
    agg = jnp.sum(e, axis=2).reshape(_R, _H) - diag
    agg = (jnp.dot(agg, we3[...], preferred_element_type=f32)
           + (_T - 1.0) * be3[...])

    fvec = jnp.broadcast_to(fg_ref[:, :_H].reshape(_S, 1, _H), (_S, _T, _H))
    fvec = fvec.reshape(_R, _H)
    nh = (jnp.dot(node, wn1a[...], preferred_element_type=f32)
          + jnp.dot(agg, wn1c[...], preferred_element_type=f32)
          + fvec + bn1[...])
    nh = jax.nn.relu(nh)
    nh = jnp.dot(nh, wn2[...], preferred_element_type=f32) + bn2[...]
    nh = nh - jnp.dot(nh, jmat, preferred_element_type=f32)
    v = jnp.dot(nh * nh, jmat, preferred_element_type=f32)
    nh = nh * (gn[...] * lax.rsqrt(v + 1e-5)) + bgn[...]
    nh = jax.nn.relu(nh)
    delta = jnp.dot(nh, wn3[...], preferred_element_type=f32) + bn3[...]
    nxt = node + delta

    hp = jnp.tanh(jnp.dot(nxt, wp0[...], preferred_element_type=f32) + bp0[...])
    logit = jnp.dot(hp, wp1[...], preferred_element_type=f32) + bp1[...]
    out_ref[...] = logit.reshape(_S, _T, _N)


def _const_spec(shape):
    return pl.BlockSpec(shape, lambda i: (0,) * len(shape))


def _build_main(interpret=False):
    weight_shapes = [
        (_N, _QH), (1, _QH), (_QH, 2 * _D), (1, 2 * _D),
        (_D, _H), (_D, _H), (1, _H), (_H, _H), (1, _H),
        (1, _H), (1, _H), (_H, _H), (1, _H),
        (_D, _H), (_H, _H), (1, _H), (_H, _H), (1, _H),
        (1, _H), (1, _H), (_H, _D), (1, _D),
        (_D, _QH), (1, _QH), (_QH, _N), (1, _N),
    ]
    in_specs = [
        pl.BlockSpec((_S, _T, _N), lambda i: (i, 0, 0)),
        pl.BlockSpec((_S, 128), lambda i: (i, 0)),
    ] + [_const_spec(s) for s in weight_shapes]
    out_specs = [
        pl.BlockSpec((_S, _T, _N), lambda i: (i, 0, 0)),
        pl.BlockSpec((1, 1, 128), lambda i: (i, 0, 0)),
    ]
    return pl.pallas_call(
        _main_body,
        grid=(_GRID,),
        in_specs=in_specs,
        out_specs=out_specs,
        out_shape=[
            jax.ShapeDtypeStruct((_B, _T, _N), jnp.float32),
            jax.ShapeDtypeStruct((_GRID, 1, 128), jnp.float32),
        ],
        compiler_params=pltpu.CompilerParams(
            dimension_semantics=("parallel",)),
        interpret=interpret,
    )


@functools.lru_cache(maxsize=None)
def _sc_gather_fn():
    info = plsc.get_sparse_core_info()
    nw = info.num_cores * info.num_subcores
    bpw = _B // nw
    mesh = plsc.VectorSubcoreMesh(core_axis_name="c", subcore_axis_name="s")

    @functools.partial(
        pl.kernel,
        mesh=mesh,
        out_type=jax.ShapeDtypeStruct((_B, 128), jnp.float32),
        scratch_types=[
            pltpu.VMEM((bpw,), jnp.int32),
            pltpu.VMEM((bpw, 128), jnp.float32),
            pltpu.SemaphoreType.DMA,
        ],
    )
    def gather_k(table_hbm, idx_hbm, out_hbm, idx_v, rows_v, sem):
        wid = lax.axis_index("s") * info.num_cores + lax.axis_index("c")
        base = wid * bpw
        pltpu.sync_copy(idx_hbm.at[pl.ds(base, bpw)], idx_v)
        pltpu.async_copy(table_hbm.at[idx_v], rows_v, sem).wait()
        pltpu.sync_copy(rows_v, out_hbm.at[pl.ds(base, bpw)])

    return gather_k


def _row(v):
    return v.reshape(1, -1)


def _assemble_args(x, p, fg):
    return (
        x, fg,
        p['W_q0'], _row(p['b_q0']), p['W_q1'], _row(p['b_q1']),
        p['We1'][:_D], p['We1'][_D:], _row(p['be1']),
        p['We2'].astype(jnp.bfloat16), _row(p['be2']), _row(p['ge']), _row(p['bge']),
        p['We3'], _row(p['be3']),
        p['Wn1'][:_D], p['Wn1'][_D + _FDIM:], _row(p['bn1']),
        p['Wn2'], _row(p['bn2']), _row(p['gn']), _row(p['bgn']),
        p['Wn3'], _row(p['bn3']),
        p['W_p0'], _row(p['b_p0']), p['W_p1'], _row(p['b_p1']),
    )


def kernel(x, params, feature, edge_index):
    p = params
    table = jnp.pad(p['Wn1'][_D:_D + _FDIM], ((0, 0), (0, 128 - _H)))
    fg = _sc_gather_fn()(table, feature)
    logits, klp = _build_main()(*_assemble_args(x, p, fg))
    kl = 0.5 * jnp.sum(klp[:, 0, 0]) / (_B * _T)
    return logits, kl


# speedup vs baseline: 1.0367x; 1.0367x over previous
import functools

import jax
import jax.numpy as jnp
from jax import lax
from jax.experimental import pallas as pl
from jax.experimental.pallas import tpu as pltpu
from jax.experimental.pallas import tpu_sc as plsc

_B, _T, _N = 1024, 20, 2048
_D, _H, _FDIM, _QH = 64, 64, 1000, 256
_S = 32
_SC = 8
_R = _S * _T
_GRID = _B // _S


def _main_body(x_ref, fg_ref,
               wq0, bq0, wq1, bq1,
               we1a, we1b, be1, we2, be2, ge, bge, we3, be3,
               wn1a, wn1c, bn1, wn2, bn2, gn, bgn, wn3, bn3,
               wp0, bp0, wp1, bp1,
               out_ref, kl_ref):
    f32 = jnp.float32

    xb = x_ref[...].reshape(_R, _N)
    nrm = jnp.sqrt(jnp.sum(xb * xb, axis=1, keepdims=True))
    inv = 1.0 / jnp.maximum(nrm, 1e-6)
    h = jnp.dot(xb, wq0[...], preferred_element_type=f32)
    h = jnp.tanh(h * inv + bq0[...])
    h = jnp.dot(h, wq1[...], preferred_element_type=f32) + bq1[...]
    mu = h[:, :_D]
    logvar = h[:, _D:]
    klp = jnp.sum(-logvar + jnp.exp(logvar) + mu * mu - 1.0)
    kl_ref[...] = jnp.broadcast_to(klp.reshape(1, 1, 1), (1, 1, 128))

    node = mu
    a = jnp.dot(node, we1a[...], preferred_element_type=f32)
    b = jnp.dot(node, we1b[...], preferred_element_type=f32)
    aa = a
    bb = b + be1[...]
    e = jax.nn.relu(aa.reshape(_S, _T, 1, _H) + bb.reshape(_S, 1, _T, _H))
    e = e.reshape(_S * _T * _T, _H)
    jmat = jnp.full((_H, _H), 1.0 / _H, dtype=f32)

    def _edge_tail(t):
        t = jnp.dot(t, we2[...], preferred_element_type=f32) + be2[...]
        t = t - jnp.dot(t, jmat, preferred_element_type=f32)
        vv = jnp.dot(t * t, jmat, preferred_element_type=f32)
        t = t * (ge[...] * lax.rsqrt(vv + 1e-5)) + bge[...]
        return jax.nn.relu(t)

    e = _edge_tail(e).reshape(_S, _T, _T, _H)

    diag = _edge_tail(jax.nn.relu(aa + bb))
    agg = jnp.sum(e, axis=2).reshape(_R, _H) - diag
    agg = (jnp.dot(agg, we3[...], preferred_element_type=f32)
           + (_T - 1.0) * be3[...])

    fvec = jnp.broadcast_to(fg_ref[:, :_H].reshape(_S, 1, _H), (_S, _T, _H))
    fvec = fvec.reshape(_R, _H)
    nh = (jnp.dot(node, wn1a[...], preferred_element_type=f32)
          + jnp.dot(agg, wn1c[...], preferred_element_type=f32)
          + fvec + bn1[...])
    nh = jax.nn.relu(nh)
    nh = jnp.dot(nh, wn2[...], preferred_element_type=f32) + bn2[...]
    nh = nh - jnp.dot(nh, jmat, preferred_element_type=f32)
    v = jnp.dot(nh * nh, jmat, preferred_element_type=f32)
    nh = nh * (gn[...] * lax.rsqrt(v + 1e-5)) + bgn[...]
    nh = jax.nn.relu(nh)
    delta = jnp.dot(nh, wn3[...], preferred_element_type=f32) + bn3[...]
    nxt = node + delta

    hp = jnp.tanh(jnp.dot(nxt, wp0[...], preferred_element_type=f32) + bp0[...])
    logit = jnp.dot(hp, wp1[...], preferred_element_type=f32) + bp1[...]
    out_ref[...] = logit.reshape(_S, _T, _N)


def _const_spec(shape):
    return pl.BlockSpec(shape, lambda i: (0,) * len(shape))


def _build_main(interpret=False):
    weight_shapes = [
        (_N, _QH), (1, _QH), (_QH, 2 * _D), (1, 2 * _D),
        (_D, _H), (_D, _H), (1, _H), (_H, _H), (1, _H),
        (1, _H), (1, _H), (_H, _H), (1, _H),
        (_D, _H), (_H, _H), (1, _H), (_H, _H), (1, _H),
        (1, _H), (1, _H), (_H, _D), (1, _D),
        (_D, _QH), (1, _QH), (_QH, _N), (1, _N),
    ]
    in_specs = [
        pl.BlockSpec((_S, _T, _N), lambda i: (i, 0, 0)),
        pl.BlockSpec((_S, 128), lambda i: (i, 0)),
    ] + [_const_spec(s) for s in weight_shapes]
    out_specs = [
        pl.BlockSpec((_S, _T, _N), lambda i: (i, 0, 0)),
        pl.BlockSpec((1, 1, 128), lambda i: (i, 0, 0)),
    ]
    return pl.pallas_call(
        _main_body,
        grid=(_GRID,),
        in_specs=in_specs,
        out_specs=out_specs,
        out_shape=[
            jax.ShapeDtypeStruct((_B, _T, _N), jnp.float32),
            jax.ShapeDtypeStruct((_GRID, 1, 128), jnp.float32),
        ],
        compiler_params=pltpu.CompilerParams(
            dimension_semantics=("parallel",)),
        interpret=interpret,
    )


@functools.lru_cache(maxsize=None)
def _sc_gather_fn():
    info = plsc.get_sparse_core_info()
    nw = info.num_cores * info.num_subcores
    bpw = _B // nw
    mesh = plsc.VectorSubcoreMesh(core_axis_name="c", subcore_axis_name="s")

    @functools.partial(
        pl.kernel,
        mesh=mesh,
        out_type=jax.ShapeDtypeStruct((_B, 128), jnp.float32),
        scratch_types=[
            pltpu.VMEM((bpw,), jnp.int32),
            pltpu.VMEM((bpw, 128), jnp.float32),
            pltpu.SemaphoreType.DMA,
        ],
    )
    def gather_k(table_hbm, idx_hbm, out_hbm, idx_v, rows_v, sem):
        wid = lax.axis_index("s") * info.num_cores + lax.axis_index("c")
        base = wid * bpw
        pltpu.sync_copy(idx_hbm.at[pl.ds(base, bpw)], idx_v)
        pltpu.async_copy(table_hbm.at[idx_v], rows_v, sem).wait()
        pltpu.sync_copy(rows_v, out_hbm.at[pl.ds(base, bpw)])

    return gather_k


def _row(v):
    return v.reshape(1, -1)


def _assemble_args(x, p, fg):
    return (
        x, fg,
        p['W_q0'], _row(p['b_q0']), p['W_q1'], _row(p['b_q1']),
        p['We1'][:_D], p['We1'][_D:], _row(p['be1']),
        p['We2'], _row(p['be2']), _row(p['ge']), _row(p['bge']),
        p['We3'], _row(p['be3']),
        p['Wn1'][:_D], p['Wn1'][_D + _FDIM:], _row(p['bn1']),
        p['Wn2'], _row(p['bn2']), _row(p['gn']), _row(p['bgn']),
        p['Wn3'], _row(p['bn3']),
        p['W_p0'], _row(p['b_p0']), p['W_p1'], _row(p['b_p1']),
    )


def kernel(x, params, feature, edge_index):
    p = params
    table = jnp.pad(p['Wn1'][_D:_D + _FDIM], ((0, 0), (0, 128 - _H)))
    fg = _sc_gather_fn()(table, feature)
    logits, klp = _build_main()(*_assemble_args(x, p, fg))
    kl = 0.5 * jnp.sum(klp[:, 0, 0]) / (_B * _T)
    return logits, kl


# speedup vs baseline: 1.0619x; 1.0242x over previous
import functools

import jax
import jax.numpy as jnp
from jax import lax
from jax.experimental import pallas as pl
from jax.experimental.pallas import tpu as pltpu
from jax.experimental.pallas import tpu_sc as plsc

_B, _T, _N = 1024, 20, 2048
_D, _H, _FDIM, _QH = 64, 64, 1000, 256
_S = 32
_SC = 8
_R = _S * _T
_GRID = _B // _S


def _main_body(x_ref, fg_ref,
               wq0, bq0, wq1, bq1,
               we1a, we1b, be1, we2, be2, ge, bge, we3, be3,
               wn1a, wn1c, bn1, wn2, bn2, gn, bgn, wn3, bn3,
               wp0, bp0, wp1, bp1,
               out_ref, kl_ref):
    f32 = jnp.float32

    xb = x_ref[...].reshape(_R, _N)
    nrm = jnp.sqrt(jnp.sum(xb * xb, axis=1, keepdims=True))
    inv = 1.0 / jnp.maximum(nrm, 1e-6)
    h = jnp.dot(xb, wq0[...], preferred_element_type=f32)
    h = jnp.tanh(h * inv + bq0[...])
    h = jnp.dot(h, wq1[...], preferred_element_type=f32) + bq1[...]
    mu = h[:, :_D]
    logvar = h[:, _D:]
    klp = jnp.sum(-logvar + jnp.exp(logvar) + mu * mu - 1.0)
    kl_ref[...] = jnp.broadcast_to(klp.reshape(1, 1, 1), (1, 1, 128))

    node = mu
    a = jnp.dot(node, we1a[...], preferred_element_type=f32)
    b = jnp.dot(node, we1b[...], preferred_element_type=f32)
    aa = a
    bb = b + be1[...]
    e = jax.nn.relu(bb.reshape(_S, _T, 1, _H) + aa.reshape(_S, 1, _T, _H))
    e = e.reshape(_S * _T * _T, _H)
    jmat = jnp.full((_H, _H), 1.0 / _H, dtype=f32)

    def _edge_tail(t):
        t = jnp.dot(t, we2[...], preferred_element_type=f32) + be2[...]
        t = t - jnp.dot(t, jmat, preferred_element_type=f32)
        vv = jnp.dot(t * t, jmat, preferred_element_type=f32)
        t = t * (ge[...] * lax.rsqrt(vv + 1e-5)) + bge[...]
        return jax.nn.relu(t)

    e = _edge_tail(e).reshape(_S, _T, _T, _H)

    diag = _edge_tail(jax.nn.relu(aa + bb))
    agg = jnp.sum(e, axis=1).reshape(_R, _H) - diag
    agg = (jnp.dot(agg, we3[...], preferred_element_type=f32)
           + (_T - 1.0) * be3[...])

    fvec = jnp.broadcast_to(fg_ref[:, :_H].reshape(_S, 1, _H), (_S, _T, _H))
    fvec = fvec.reshape(_R, _H)
    nh = (jnp.dot(node, wn1a[...], preferred_element_type=f32)
          + jnp.dot(agg, wn1c[...], preferred_element_type=f32)
          + fvec + bn1[...])
    nh = jax.nn.relu(nh)
    nh = jnp.dot(nh, wn2[...], preferred_element_type=f32) + bn2[...]
    nh = nh - jnp.dot(nh, jmat, preferred_element_type=f32)
    v = jnp.dot(nh * nh, jmat, preferred_element_type=f32)
    nh = nh * (gn[...] * lax.rsqrt(v + 1e-5)) + bgn[...]
    nh = jax.nn.relu(nh)
    delta = jnp.dot(nh, wn3[...], preferred_element_type=f32) + bn3[...]
    nxt = node + delta

    hp = jnp.tanh(jnp.dot(nxt, wp0[...], preferred_element_type=f32) + bp0[...])
    logit = jnp.dot(hp, wp1[...], preferred_element_type=f32) + bp1[...]
    out_ref[...] = logit.reshape(_S, _T, _N)


def _const_spec(shape):
    return pl.BlockSpec(shape, lambda i: (0,) * len(shape))


def _build_main(interpret=False):
    weight_shapes = [
        (_N, _QH), (1, _QH), (_QH, 2 * _D), (1, 2 * _D),
        (_D, _H), (_D, _H), (1, _H), (_H, _H), (1, _H),
        (1, _H), (1, _H), (_H, _H), (1, _H),
        (_D, _H), (_H, _H), (1, _H), (_H, _H), (1, _H),
        (1, _H), (1, _H), (_H, _D), (1, _D),
        (_D, _QH), (1, _QH), (_QH, _N), (1, _N),
    ]
    in_specs = [
        pl.BlockSpec((_S, _T, _N), lambda i: (i, 0, 0)),
        pl.BlockSpec((_S, 128), lambda i: (i, 0)),
    ] + [_const_spec(s) for s in weight_shapes]
    out_specs = [
        pl.BlockSpec((_S, _T, _N), lambda i: (i, 0, 0)),
        pl.BlockSpec((1, 1, 128), lambda i: (i, 0, 0)),
    ]
    return pl.pallas_call(
        _main_body,
        grid=(_GRID,),
        in_specs=in_specs,
        out_specs=out_specs,
        out_shape=[
            jax.ShapeDtypeStruct((_B, _T, _N), jnp.float32),
            jax.ShapeDtypeStruct((_GRID, 1, 128), jnp.float32),
        ],
        compiler_params=pltpu.CompilerParams(
            dimension_semantics=("parallel",)),
        interpret=interpret,
    )


@functools.lru_cache(maxsize=None)
def _sc_gather_fn():
    info = plsc.get_sparse_core_info()
    nw = info.num_cores * info.num_subcores
    bpw = _B // nw
    mesh = plsc.VectorSubcoreMesh(core_axis_name="c", subcore_axis_name="s")

    @functools.partial(
        pl.kernel,
        mesh=mesh,
        out_type=jax.ShapeDtypeStruct((_B, 128), jnp.float32),
        scratch_types=[
            pltpu.VMEM((bpw,), jnp.int32),
            pltpu.VMEM((bpw, 128), jnp.float32),
            pltpu.SemaphoreType.DMA,
        ],
    )
    def gather_k(table_hbm, idx_hbm, out_hbm, idx_v, rows_v, sem):
        wid = lax.axis_index("s") * info.num_cores + lax.axis_index("c")
        base = wid * bpw
        pltpu.sync_copy(idx_hbm.at[pl.ds(base, bpw)], idx_v)
        pltpu.async_copy(table_hbm.at[idx_v], rows_v, sem).wait()
        pltpu.sync_copy(rows_v, out_hbm.at[pl.ds(base, bpw)])

    return gather_k


def _row(v):
    return v.reshape(1, -1)


def _assemble_args(x, p, fg):
    return (
        x, fg,
        p['W_q0'], _row(p['b_q0']), p['W_q1'], _row(p['b_q1']),
        p['We1'][:_D], p['We1'][_D:], _row(p['be1']),
        p['We2'], _row(p['be2']), _row(p['ge']), _row(p['bge']),
        p['We3'], _row(p['be3']),
        p['Wn1'][:_D], p['Wn1'][_D + _FDIM:], _row(p['bn1']),
        p['Wn2'], _row(p['bn2']), _row(p['gn']), _row(p['bgn']),
        p['Wn3'], _row(p['bn3']),
        p['W_p0'], _row(p['b_p0']), p['W_p1'], _row(p['b_p1']),
    )


def kernel(x, params, feature, edge_index):
    p = params
    table = jnp.pad(p['Wn1'][_D:_D + _FDIM], ((0, 0), (0, 128 - _H)))
    fg = _sc_gather_fn()(table, feature)
    logits, klp = _build_main()(*_assemble_args(x, p, fg))
    kl = 0.5 * jnp.sum(klp[:, 0, 0]) / (_B * _T)
    return logits, kl
